# baseline (speedup 1.0000x reference)
import functools

import jax
import jax.numpy as jnp
from jax import lax
from jax.experimental import pallas as pl
from jax.experimental.pallas import tpu as pltpu
from jax.experimental.pallas import tpu_sc as plsc

_LANES = 16
_CHUNK = 128


_TB = 32768
_CS = 16


def _transform2_body(c, cs, before_ref, after_ref, w1_ref, b1_ref, w2_ref, b2_ref, f_ref):
    k = pl.program_id(2)
    x = before_ref[0]
    y = after_ref[0]
    w1a = w1_ref[0, 0]
    w1b = w1_ref[1, 0]
    w2 = w2_ref[...]
    a = jnp.dot(w1a, w2, preferred_element_type=jnp.float32)
    bm = jnp.dot(w1b, w2, preferred_element_type=jnp.float32)
    part = lax.dot_general(x, a, (((0,), (0,)), ((), ())),
                           preferred_element_type=jnp.float32)
    part = part + lax.dot_general(y, bm, (((0,), (0,)), ((), ())),
                                  preferred_element_type=jnp.float32)

    @pl.when(k == 0)
    def _():
        bias = jnp.dot(b1_ref[...], w2, preferred_element_type=jnp.float32) + b2_ref[...]
        f_ref[0] = part + bias[None, :]

    @pl.when(k > 0)
    def _():
        f_ref[0] += part


def _transform2(before2, after2, w1, b1, w2p, b2p):
    b, c, hw = before2.shape
    w1r = w1.reshape(2, c // _CS, _CS, 32)
    grid = (b, hw // _TB, c // _CS)
    return pl.pallas_call(
        functools.partial(_transform2_body, c, _CS),
        grid=grid,
        in_specs=[
            pl.BlockSpec((1, _CS, _TB), lambda i, j, k: (i, k, j)),
            pl.BlockSpec((1, _CS, _TB), lambda i, j, k: (i, k, j)),
            pl.BlockSpec((2, 1, _CS, 32), lambda i, j, k: (0, k, 0, 0)),
            pl.BlockSpec((32,), lambda i, j, k: (0,)),
            pl.BlockSpec((32, _LANES), lambda i, j, k: (0, 0)),
            pl.BlockSpec((_LANES,), lambda i, j, k: (0,)),
        ],
        out_specs=pl.BlockSpec((1, _TB, _LANES), lambda i, j, k: (i, j, 0)),
        out_shape=jax.ShapeDtypeStruct((b, hw, _LANES), jnp.float32),
    )(before2, after2, w1r, b1, w2p, b2p)


_KBUF = 7


def _make_gather(total, n_per_batch, hw, w, nbatch):
    info = plsc.get_sparse_core_info()
    nw = info.num_cores * info.num_subcores
    cpw = -(-total // (nw * _CHUNK)) * _CHUNK
    nchunk = cpw // _CHUNK
    npair = -(-nchunk // (2 * _KBUF))
    nchunk = npair * 2 * _KBUF
    cpw = nchunk * _CHUNK
    mesh = plsc.VectorSubcoreMesh(core_axis_name="c", subcore_axis_name="s")

    @functools.partial(
        pl.kernel,
        mesh=mesh,
        compiler_params=pltpu.CompilerParams(use_tc_tiling_on_sc=False),
        out_type=jax.ShapeDtypeStruct((total, _LANES), jnp.float32),
        scratch_types=[
            pltpu.VMEM((cpw,), jnp.int32),
            pltpu.VMEM((cpw,), jnp.int32),
            pltpu.VMEM((nchunk, _CHUNK), jnp.int32),
            pltpu.VMEM((_KBUF, _CHUNK, _LANES), jnp.float32),
            pltpu.VMEM((_KBUF, _CHUNK, _LANES), jnp.float32),
            pltpu.SemaphoreType.DMA,
            pltpu.SemaphoreType.DMA,
        ],
    )
    def gather_kernel(f_hbm, y_hbm, x_hbm, out_hbm, y_v, x_v, idx_v, r_a, r_b, s_a, s_b):
        wid = lax.axis_index("s") * info.num_cores + lax.axis_index("c")
        base = lax.min(wid * cpw, total - cpw)
        pltpu.sync_copy(y_hbm.at[pl.ds(base, cpw)], y_v)
        pltpu.sync_copy(x_hbm.at[pl.ds(base, cpw)], x_v)

        lanes = lax.iota(jnp.int32, 16)
        nvec = jnp.full((16,), n_per_batch, jnp.int32)
        bmax = jnp.full((16,), nbatch - 1, jnp.int32)

        def chunk_idx_body(j, _):
            def lane_body(g, _):
                off = j * _CHUNK + g * 16
                yy = y_v[pl.ds(off, 16)]
                xx = x_v[pl.ds(off, 16)]
                pos = base + off + lanes
                bidx = lax.min(lax.div(pos, nvec), bmax)
                idx_v[j, pl.ds(g * 16, 16)] = bidx * hw + yy * w + xx
                return 0
            lax.fori_loop(0, _CHUNK // 16, lane_body, 0)
            return 0

        lax.fori_loop(0, nchunk, chunk_idx_body, 0)

        def fire(grp, rbuf, sem):
            for kb in range(_KBUF):
                pltpu.async_copy(f_hbm.at[idx_v.at[grp * _KBUF + kb]], rbuf.at[kb], sem)

        def drain_store(grp, rbuf, sem):
            for kb in range(_KBUF):
                chunk = grp * _KBUF + kb
                pltpu.make_async_copy(f_hbm.at[idx_v.at[chunk]], rbuf.at[kb], sem).wait()
                pltpu.sync_copy(rbuf.at[kb], out_hbm.at[pl.ds(base + chunk * _CHUNK, _CHUNK)])

        fire(0, r_a, s_a)

        def pair_body(t, _):
            g0 = 2 * t
            fire(g0 + 1, r_b, s_b)
            drain_store(g0, r_a, s_a)

            @pl.when(g0 + 2 < nchunk // _KBUF)
            def _():
                fire(g0 + 2, r_a, s_a)

            drain_store(g0 + 1, r_b, s_b)
            return 0

        lax.fori_loop(0, npair, pair_body, 0)

    return gather_kernel


def kernel(before_pseudoimages, after_pseudoimages, points, voxel_coords, W1, b1, W2, b2):
    b, c, h, w = before_pseudoimages.shape
    n = voxel_coords.shape[1]
    hw = h * w

    w2p = jnp.zeros((W2.shape[0], _LANES), jnp.float32).at[:, : W2.shape[1]].set(W2)
    b2p = jnp.zeros((_LANES,), jnp.float32).at[: b2.shape[0]].set(b2)

    f = _transform2(
        before_pseudoimages.reshape(b, c, hw),
        after_pseudoimages.reshape(b, c, hw),
        W1, b1, w2p, b2p,
    )
    f_flat = f.reshape(b * hw, _LANES)

    total = b * n
    yf = voxel_coords[:, :, 1].reshape(-1).astype(jnp.int32)
    xf = voxel_coords[:, :, 2].reshape(-1).astype(jnp.int32)

    gathered = _make_gather(total, n, hw, w, b)(f_flat, yf, xf)
    return gathered[:, :3].reshape(b, n, 3)


# speedup vs baseline: 2.5838x; 2.5838x over previous
import functools

import jax
import jax.numpy as jnp
from jax import lax
from jax.experimental import pallas as pl
from jax.experimental.pallas import tpu as pltpu
from jax.experimental.pallas import tpu_sc as plsc

_CHUNK = 128
_KBUF = 7
_HB = 16


def _transform_body(c, before_ref, after_ref, w1_ref, b1_ref, w2_ref, b2_ref, f_ref):
    cs, hb, w = before_ref.shape[1:]
    x = before_ref[0].reshape(cs, hb * w)
    y = after_ref[0].reshape(cs, hb * w)
    w1 = w1_ref[...]
    w2 = w2_ref[...]
    a = jnp.dot(w1[:c], w2, preferred_element_type=jnp.float32)
    bm = jnp.dot(w1[c:], w2, preferred_element_type=jnp.float32)
    bias = jnp.dot(b1_ref[...], w2, preferred_element_type=jnp.float32) + b2_ref[...]
    part = lax.dot_general(x, a, (((0,), (0,)), ((), ())),
                           preferred_element_type=jnp.float32)
    part = part + lax.dot_general(y, bm, (((0,), (0,)), ((), ())),
                                  preferred_element_type=jnp.float32)
    f_ref[...] = part + bias[None, :]


def _transform(before, after, w1, b1, w2, b2):
    b, c, h, w = before.shape
    tb = _HB * w
    grid = (b, h // _HB)
    return pl.pallas_call(
        functools.partial(_transform_body, c),
        grid=grid,
        in_specs=[
            pl.BlockSpec((1, c, _HB, w), lambda i, j: (i, 0, j, 0)),
            pl.BlockSpec((1, c, _HB, w), lambda i, j: (i, 0, j, 0)),
            pl.BlockSpec((2 * c, 32), lambda i, j: (0, 0)),
            pl.BlockSpec((32,), lambda i, j: (0,)),
            pl.BlockSpec((32, 16), lambda i, j: (0, 0)),
            pl.BlockSpec((16,), lambda i, j: (0,)),
        ],
        out_specs=pl.BlockSpec((tb, 16), lambda i, j: (i * (h // _HB) + j, 0)),
        out_shape=jax.ShapeDtypeStruct((b * h * w, 16), jnp.float32),
    )(before, after, w1, b1, w2, b2)


def _make_gather(total, n_per_batch, hw, w, nbatch):
    info = plsc.get_sparse_core_info()
    nw = info.num_cores * info.num_subcores
    cpw = -(-total // (nw * _CHUNK)) * _CHUNK
    nchunk = cpw // _CHUNK
    npair = -(-nchunk // (2 * _KBUF))
    nchunk = npair * 2 * _KBUF
    cpw = nchunk * _CHUNK
    mesh = plsc.VectorSubcoreMesh(core_axis_name="c", subcore_axis_name="s")

    @functools.partial(
        pl.kernel,
        mesh=mesh,
        compiler_params=pltpu.CompilerParams(use_tc_tiling_on_sc=False),
        out_type=jax.ShapeDtypeStruct((total, 16), jnp.float32),
        scratch_types=[
            pltpu.VMEM((cpw,), jnp.int32),
            pltpu.VMEM((cpw,), jnp.int32),
            pltpu.VMEM((nchunk, _CHUNK), jnp.int32),
            pltpu.VMEM((_KBUF, _CHUNK, 16), jnp.float32),
            pltpu.VMEM((_KBUF, _CHUNK, 16), jnp.float32),
            pltpu.SemaphoreType.DMA,
            pltpu.SemaphoreType.DMA,
        ],
    )
    def gather_kernel(f_hbm, y_hbm, x_hbm, out_hbm, y_v, x_v, idx_v, r_a, r_b, s_a, s_b):
        wid = lax.axis_index("s") * info.num_cores + lax.axis_index("c")
        base = lax.min(wid * cpw, total - cpw)
        pltpu.sync_copy(y_hbm.at[pl.ds(base, cpw)], y_v)
        pltpu.sync_copy(x_hbm.at[pl.ds(base, cpw)], x_v)

        lanes = lax.iota(jnp.int32, 16)
        nvec = jnp.full((16,), n_per_batch, jnp.int32)
        bmax = jnp.full((16,), nbatch - 1, jnp.int32)

        def chunk_idx_body(j, _):
            def lane_body(g, _):
                off = j * _CHUNK + g * 16
                yy = y_v[pl.ds(off, 16)]
                xx = x_v[pl.ds(off, 16)]
                pos = base + off + lanes
                bidx = lax.min(lax.div(pos, nvec), bmax)
                idx_v[j, pl.ds(g * 16, 16)] = bidx * hw + yy * w + xx
                return 0
            lax.fori_loop(0, _CHUNK // 16, lane_body, 0)
            return 0

        lax.fori_loop(0, nchunk, chunk_idx_body, 0)

        def fire(grp, rbuf, sem):
            for kb in range(_KBUF):
                pltpu.async_copy(f_hbm.at[idx_v.at[grp * _KBUF + kb]], rbuf.at[kb], sem)

        def drain_store(grp, rbuf, sem):
            for kb in range(_KBUF):
                chunk = grp * _KBUF + kb
                pltpu.make_async_copy(f_hbm.at[idx_v.at[chunk]], rbuf.at[kb], sem).wait()
                pltpu.sync_copy(rbuf.at[kb], out_hbm.at[pl.ds(base + chunk * _CHUNK, _CHUNK)])

        fire(0, r_a, s_a)

        def pair_body(t, _):
            g0 = 2 * t
            fire(g0 + 1, r_b, s_b)
            drain_store(g0, r_a, s_a)

            @pl.when(g0 + 2 < nchunk // _KBUF)
            def _():
                fire(g0 + 2, r_a, s_a)

            drain_store(g0 + 1, r_b, s_b)
            return 0

        lax.fori_loop(0, npair, pair_body, 0)

    return gather_kernel


def kernel(before_pseudoimages, after_pseudoimages, points, voxel_coords, W1, b1, W2, b2):
    b, c, h, w = before_pseudoimages.shape
    n = voxel_coords.shape[1]
    hw = h * w

    w2p = jnp.zeros((W2.shape[0], 16), jnp.float32).at[:, : W2.shape[1]].set(W2)
    b2p = jnp.zeros((16,), jnp.float32).at[: b2.shape[0]].set(b2)
    f = _transform(before_pseudoimages, after_pseudoimages, W1, b1, w2p, b2p)

    total = b * n
    yf = voxel_coords[:, :, 1].reshape(-1).astype(jnp.int32)
    xf = voxel_coords[:, :, 2].reshape(-1).astype(jnp.int32)

    gathered = _make_gather(total, n, hw, w, b)(f, yf, xf)
    return gathered[:, :3].reshape(b, n, 3)


# speedup vs baseline: 3.3260x; 1.2873x over previous
import functools

import jax
import jax.numpy as jnp
from jax import lax
from jax.experimental import pallas as pl
from jax.experimental.pallas import tpu as pltpu
from jax.experimental.pallas import tpu_sc as plsc

_CHUNK = 128
_KBUF = 7
_HB = 16


def _transform_body(c, before_ref, after_ref, w1_ref, b1_ref, w2_ref, b2_ref, f_ref):
    cs, hb, w = before_ref.shape[1:]
    x = before_ref[0].reshape(cs, hb * w)
    y = after_ref[0].reshape(cs, hb * w)
    w1 = w1_ref[...]
    w2 = w2_ref[...]
    a = jnp.dot(w1[:c], w2, preferred_element_type=jnp.float32)
    bm = jnp.dot(w1[c:], w2, preferred_element_type=jnp.float32)
    bias = jnp.dot(b1_ref[...], w2, preferred_element_type=jnp.float32) + b2_ref[...]
    part = lax.dot_general(x, a, (((0,), (0,)), ((), ())),
                           preferred_element_type=jnp.float32)
    part = part + lax.dot_general(y, bm, (((0,), (0,)), ((), ())),
                                  preferred_element_type=jnp.float32)
    out = part + bias[None, :]
    out3 = out.reshape(hb * w // 8, 8, 16)
    for s in range(8):
        f_ref[:, pl.ds(s * 16, 16)] = out3[:, s, :]


def _transform(before, after, w1, b1, w2, b2):
    b, c, h, w = before.shape
    tb = _HB * w
    grid = (b, h // _HB)
    return pl.pallas_call(
        functools.partial(_transform_body, c),
        grid=grid,
        in_specs=[
            pl.BlockSpec((1, c, _HB, w), lambda i, j: (i, 0, j, 0)),
            pl.BlockSpec((1, c, _HB, w), lambda i, j: (i, 0, j, 0)),
            pl.BlockSpec((2 * c, 32), lambda i, j: (0, 0)),
            pl.BlockSpec((32,), lambda i, j: (0,)),
            pl.BlockSpec((32, 16), lambda i, j: (0, 0)),
            pl.BlockSpec((16,), lambda i, j: (0,)),
        ],
        out_specs=pl.BlockSpec((tb // 8, 128), lambda i, j: (i * (h // _HB) + j, 0)),
        out_shape=jax.ShapeDtypeStruct((b * h * w // 8, 128), jnp.float32),
    )(before, after, w1, b1, w2, b2)


def _make_gather(total, n_per_batch, hw, w, nbatch):
    info = plsc.get_sparse_core_info()
    nw = info.num_cores * info.num_subcores
    cpw = -(-total // (nw * _CHUNK)) * _CHUNK
    nchunk = cpw // _CHUNK
    npair = -(-nchunk // (2 * _KBUF))
    nchunk = npair * 2 * _KBUF
    cpw = nchunk * _CHUNK
    mesh = plsc.VectorSubcoreMesh(core_axis_name="c", subcore_axis_name="s")

    @functools.partial(
        pl.kernel,
        mesh=mesh,
        compiler_params=pltpu.CompilerParams(use_tc_tiling_on_sc=False),
        out_type=jax.ShapeDtypeStruct((total, 16), jnp.float32),
        scratch_types=[
            pltpu.VMEM((cpw,), jnp.int32),
            pltpu.VMEM((cpw,), jnp.int32),
            pltpu.VMEM((nchunk, _CHUNK), jnp.int32),
            pltpu.VMEM((_KBUF, _CHUNK, 16), jnp.float32),
            pltpu.VMEM((_KBUF, _CHUNK, 16), jnp.float32),
            pltpu.SemaphoreType.DMA,
            pltpu.SemaphoreType.DMA,
        ],
    )
    def gather_kernel(f_hbm, y_hbm, x_hbm, out_hbm, y_v, x_v, idx_v, r_a, r_b, s_a, s_b):
        wid = lax.axis_index("s") * info.num_cores + lax.axis_index("c")
        base = lax.min(wid * cpw, total - cpw)
        pltpu.sync_copy(y_hbm.at[pl.ds(base, cpw)], y_v)
        pltpu.sync_copy(x_hbm.at[pl.ds(base, cpw)], x_v)

        lanes = lax.iota(jnp.int32, 16)
        nvec = jnp.full((16,), n_per_batch, jnp.int32)
        bmax = jnp.full((16,), nbatch - 1, jnp.int32)

        def chunk_idx_body(j, _):
            def lane_body(g, _):
                off = j * _CHUNK + g * 16
                yy = y_v[pl.ds(off, 16)]
                xx = x_v[pl.ds(off, 16)]
                pos = base + off + lanes
                bidx = lax.min(lax.div(pos, nvec), bmax)
                idx_v[j, pl.ds(g * 16, 16)] = bidx * hw + yy * w + xx
                return 0
            lax.fori_loop(0, _CHUNK // 16, lane_body, 0)
            return 0

        lax.fori_loop(0, nchunk, chunk_idx_body, 0)

        def fire(grp, rbuf, sem):
            for kb in range(_KBUF):
                pltpu.async_copy(f_hbm.at[idx_v.at[grp * _KBUF + kb]], rbuf.at[kb], sem)

        def drain_store(grp, rbuf, sem):
            for kb in range(_KBUF):
                chunk = grp * _KBUF + kb
                pltpu.make_async_copy(f_hbm.at[idx_v.at[chunk]], rbuf.at[kb], sem).wait()
                pltpu.sync_copy(rbuf.at[kb], out_hbm.at[pl.ds(base + chunk * _CHUNK, _CHUNK)])

        fire(0, r_a, s_a)

        def pair_body(t, _):
            g0 = 2 * t
            fire(g0 + 1, r_b, s_b)
            drain_store(g0, r_a, s_a)

            @pl.when(g0 + 2 < nchunk // _KBUF)
            def _():
                fire(g0 + 2, r_a, s_a)

            drain_store(g0 + 1, r_b, s_b)
            return 0

        lax.fori_loop(0, npair, pair_body, 0)

    return gather_kernel


def kernel(before_pseudoimages, after_pseudoimages, points, voxel_coords, W1, b1, W2, b2):
    b, c, h, w = before_pseudoimages.shape
    n = voxel_coords.shape[1]
    hw = h * w

    w2p = jnp.zeros((W2.shape[0], 16), jnp.float32).at[:, : W2.shape[1]].set(W2)
    b2p = jnp.zeros((16,), jnp.float32).at[: b2.shape[0]].set(b2)
    f = _transform(before_pseudoimages, after_pseudoimages, W1, b1, w2p, b2p)
    f = f.reshape(b * hw, 16)

    total = b * n
    yf = voxel_coords[:, :, 1].reshape(-1).astype(jnp.int32)
    xf = voxel_coords[:, :, 2].reshape(-1).astype(jnp.int32)

    gathered = _make_gather(total, n, hw, w, b)(f, yf, xf)
    return gathered[:, :3].reshape(b, n, 3)


# speedup vs baseline: 3.5592x; 1.0701x over previous
import functools

import jax
import jax.numpy as jnp
from jax import lax
from jax.experimental import pallas as pl
from jax.experimental.pallas import tpu as pltpu
from jax.experimental.pallas import tpu_sc as plsc

_CHUNK = 128
_KBUF = 7
_HB = 16


def _transform_body(c, before_ref, after_ref, w1_ref, b1_ref, w2_ref, b2_ref, f_ref):
    cs, hb, w = before_ref.shape[1:]
    x = before_ref[0].reshape(cs, hb * w)
    y = after_ref[0].reshape(cs, hb * w)
    xy = jnp.concatenate([x, y], axis=0)
    w2 = w2_ref[...]
    ab = jnp.dot(w1_ref[...], w2, preferred_element_type=jnp.float32)
    bias = jnp.dot(b1_ref[...], w2, preferred_element_type=jnp.float32) + b2_ref[...]
    part = lax.dot_general(xy, ab, (((0,), (0,)), ((), ())),
                           preferred_element_type=jnp.float32)
    out3 = part.reshape(hb * w // 8, 8, 16)
    packed = jnp.concatenate([out3[:, s, :] for s in range(8)], axis=1)
    f_ref[...] = packed + jnp.concatenate([bias] * 8)[None, :]


def _transform(before, after, w1, b1, w2, b2):
    b, c, h, w = before.shape
    tb = _HB * w
    grid = (b, h // _HB)
    return pl.pallas_call(
        functools.partial(_transform_body, c),
        grid=grid,
        in_specs=[
            pl.BlockSpec((1, c, _HB, w), lambda i, j: (i, 0, j, 0)),
            pl.BlockSpec((1, c, _HB, w), lambda i, j: (i, 0, j, 0)),
            pl.BlockSpec((2 * c, 32), lambda i, j: (0, 0)),
            pl.BlockSpec((32,), lambda i, j: (0,)),
            pl.BlockSpec((32, 16), lambda i, j: (0, 0)),
            pl.BlockSpec((16,), lambda i, j: (0,)),
        ],
        out_specs=pl.BlockSpec((tb // 8, 128), lambda i, j: (i * (h // _HB) + j, 0)),
        out_shape=jax.ShapeDtypeStruct((b * h * w // 8, 128), jnp.float32),
    )(before, after, w1, b1, w2, b2)


def _make_gather(total, n_per_batch, hw, w, nbatch):
    info = plsc.get_sparse_core_info()
    nw = info.num_cores * info.num_subcores
    cpw = -(-total // (nw * _CHUNK)) * _CHUNK
    nchunk = cpw // _CHUNK
    npair = -(-nchunk // (2 * _KBUF))
    nchunk = npair * 2 * _KBUF
    cpw = nchunk * _CHUNK
    mesh = plsc.VectorSubcoreMesh(core_axis_name="c", subcore_axis_name="s")

    @functools.partial(
        pl.kernel,
        mesh=mesh,
        compiler_params=pltpu.CompilerParams(use_tc_tiling_on_sc=False),
        out_type=jax.ShapeDtypeStruct((total, 16), jnp.float32),
        scratch_types=[
            pltpu.VMEM((cpw,), jnp.int32),
            pltpu.VMEM((cpw,), jnp.int32),
            pltpu.VMEM((nchunk, _CHUNK), jnp.int32),
            pltpu.VMEM((_KBUF, _CHUNK, 16), jnp.float32),
            pltpu.VMEM((_KBUF, _CHUNK, 16), jnp.float32),
            pltpu.SemaphoreType.DMA,
            pltpu.SemaphoreType.DMA,
        ],
    )
    def gather_kernel(f_hbm, y_hbm, x_hbm, out_hbm, y_v, x_v, idx_v, r_a, r_b, s_a, s_b):
        wid = lax.axis_index("s") * info.num_cores + lax.axis_index("c")
        base = lax.min(wid * cpw, total - cpw)
        pltpu.sync_copy(y_hbm.at[pl.ds(base, cpw)], y_v)
        pltpu.sync_copy(x_hbm.at[pl.ds(base, cpw)], x_v)

        lanes = lax.iota(jnp.int32, 16)
        nvec = jnp.full((16,), n_per_batch, jnp.int32)
        bmax = jnp.full((16,), nbatch - 1, jnp.int32)

        def chunk_idx_body(j, _):
            def lane_body(g, _):
                off = j * _CHUNK + g * 16
                yy = y_v[pl.ds(off, 16)]
                xx = x_v[pl.ds(off, 16)]
                pos = base + off + lanes
                bidx = lax.min(lax.div(pos, nvec), bmax)
                idx_v[j, pl.ds(g * 16, 16)] = bidx * hw + yy * w + xx
                return 0
            lax.fori_loop(0, _CHUNK // 16, lane_body, 0)
            return 0

        lax.fori_loop(0, nchunk, chunk_idx_body, 0)

        def fire(grp, rbuf, sem):
            for kb in range(_KBUF):
                pltpu.async_copy(f_hbm.at[idx_v.at[grp * _KBUF + kb]], rbuf.at[kb], sem)

        def drain_store(grp, rbuf, sem):
            for kb in range(_KBUF):
                chunk = grp * _KBUF + kb
                pltpu.make_async_copy(f_hbm.at[idx_v.at[chunk]], rbuf.at[kb], sem).wait()
                pltpu.sync_copy(rbuf.at[kb], out_hbm.at[pl.ds(base + chunk * _CHUNK, _CHUNK)])

        fire(0, r_a, s_a)

        def pair_body(t, _):
            g0 = 2 * t
            fire(g0 + 1, r_b, s_b)
            drain_store(g0, r_a, s_a)

            @pl.when(g0 + 2 < nchunk // _KBUF)
            def _():
                fire(g0 + 2, r_a, s_a)

            drain_store(g0 + 1, r_b, s_b)
            return 0

        lax.fori_loop(0, npair, pair_body, 0)

    return gather_kernel


def kernel(before_pseudoimages, after_pseudoimages, points, voxel_coords, W1, b1, W2, b2):
    b, c, h, w = before_pseudoimages.shape
    n = voxel_coords.shape[1]
    hw = h * w

    w2p = jnp.zeros((W2.shape[0], 16), jnp.float32).at[:, : W2.shape[1]].set(W2)
    b2p = jnp.zeros((16,), jnp.float32).at[: b2.shape[0]].set(b2)
    f = _transform(before_pseudoimages, after_pseudoimages, W1, b1, w2p, b2p)
    f = f.reshape(b * hw, 16)

    total = b * n
    yf = voxel_coords[:, :, 1].reshape(-1).astype(jnp.int32)
    xf = voxel_coords[:, :, 2].reshape(-1).astype(jnp.int32)

    gathered = _make_gather(total, n, hw, w, b)(f, yf, xf)
    return gathered[:, :3].reshape(b, n, 3)
